# baseline (speedup 1.0000x reference)
import functools

import jax
import jax.numpy as jnp
from jax import lax
from jax.experimental import pallas as pl
from jax.experimental.pallas import tpu as pltpu
from jax.experimental.pallas import tpu_sc as plsc

N_NODES = 10000
N_EDGES = 40000
D = 128
H = 32
ED = 16

NW = 32
CH = 128
NCH_H = 5
E_PER_W_H = NCH_H * CH
E_HALF = NW * E_PER_W_H
E_PAD = 2 * E_HALF
N_PAD = 10240
ZROWS = N_PAD // 16
OROWS = N_NODES // 16

BE = 256
BN = 1000


def _gather_body(x_hbm, col_hbm, out_hbm, idx_v, rows_v, sem0, sem1):
    c = lax.axis_index("c")
    s = lax.axis_index("s")
    wid = s * 2 + c
    base = wid * E_PER_W_H
    sems = (sem0, sem1)

    pltpu.sync_copy(col_hbm.at[wid], idx_v)
    cps = [pltpu.async_copy(x_hbm.at[idx_v.at[0]], rows_v.at[0], sem0), None]
    for j in range(NCH_H):
        b = j % 2
        if j + 1 < NCH_H:
            nb = (j + 1) % 2
            cps[nb] = pltpu.async_copy(
                x_hbm.at[idx_v.at[j + 1]], rows_v.at[nb], sems[nb])
        cps[b].wait()
        pltpu.sync_copy(rows_v.at[b], out_hbm.at[pl.ds(base + j * CH, CH)])


@functools.cache
def _gather():
    mesh = plsc.VectorSubcoreMesh(core_axis_name="c", subcore_axis_name="s")
    return pl.kernel(
        _gather_body,
        mesh=mesh,
        out_type=jax.ShapeDtypeStruct((E_HALF, D), jnp.float32),
        scratch_types=[
            pltpu.VMEM((NCH_H, CH), jnp.int32),
            pltpu.VMEM((2, CH, D), jnp.float32),
            pltpu.SemaphoreType.DMA,
            pltpu.SemaphoreType.DMA,
        ],
    )


def _edge_body(ea_ref, hj_ref, w1_ref, b1_ref, w2_ref, b2_ref, rsel_ref,
               out_ref, *, e_base):
    ea = ea_ref[...]
    hj = hj_ref[...]
    h = jnp.dot(ea, w1_ref[...], preferred_element_type=jnp.float32)
    h = h + b1_ref[...]
    h = h * jax.nn.sigmoid(h)
    a = jnp.dot(h.astype(jnp.bfloat16), w2_ref[...],
                preferred_element_type=jnp.float32)
    hj_rep = jnp.dot(hj.astype(jnp.bfloat16), rsel_ref[...],
                     preferred_element_type=jnp.float32)
    p = a * hj_rep
    w = D * H
    while w > H:
        w //= 2
        p = p[:, :w] + p[:, w:]
    m = p + jnp.dot(hj, b2_ref[...], preferred_element_type=jnp.float32)
    e0 = e_base + pl.program_id(0) * BE
    eids = e0 + lax.broadcasted_iota(jnp.int32, (BE, H), 0)
    out_ref[...] = jnp.where(eids < N_EDGES, m, 0.0)


def _edge_messages(ea, hj, W1, b1, W2, b2, Rsel, e_base):
    grid = (E_HALF // BE,)
    return pl.pallas_call(
        functools.partial(_edge_body, e_base=e_base),
        grid=grid,
        in_specs=[
            pl.BlockSpec((BE, ED), lambda i: (i, 0)),
            pl.BlockSpec((BE, D), lambda i: (i, 0)),
            pl.BlockSpec((ED, D), lambda i: (0, 0)),
            pl.BlockSpec((1, D), lambda i: (0, 0)),
            pl.BlockSpec((D, H * D), lambda i: (0, 0)),
            pl.BlockSpec((D, H), lambda i: (0, 0)),
            pl.BlockSpec((D, H * D), lambda i: (0, 0)),
        ],
        out_specs=pl.BlockSpec((BE, H), lambda i: (i, 0)),
        out_shape=jax.ShapeDtypeStruct((E_HALF, H), jnp.float32),
    )(ea, hj, W1, b1, W2, b2, Rsel)


def _scatter_body(msg0_hbm, msg1_hbm, row_hbm, zeros_hbm, out_hbm,
                  idx_v, msg_v, shared):
    c = lax.axis_index("c")
    s = lax.axis_index("s")
    wid = s * 2 + c
    pltpu.sync_copy(zeros_hbm.at[pl.ds(s * ZROWS, ZROWS)],
                    shared.at[pl.ds(s * ZROWS, ZROWS)])
    pltpu.sync_copy(row_hbm.at[wid], idx_v)
    plsc.subcore_barrier()

    for j in range(2 * NCH_H):
        src = msg0_hbm if j < NCH_H else msg1_hbm
        off = wid * E_PER_W_H + (j % NCH_H) * CH
        pltpu.sync_copy(src.at[pl.ds(off, CH)], msg_v)
        pltpu.sync_copy(msg_v, shared.at[idx_v.at[j]], add=True)

    plsc.subcore_barrier()
    pltpu.sync_copy(shared.at[pl.ds(s * OROWS, OROWS)],
                    out_hbm.at[c, pl.ds(s * OROWS, OROWS)])


@functools.cache
def _scatter():
    mesh = plsc.VectorSubcoreMesh(core_axis_name="c", subcore_axis_name="s")
    return pl.kernel(
        _scatter_body,
        mesh=mesh,
        out_type=jax.ShapeDtypeStruct((2, N_NODES, H), jnp.float32),
        scratch_types=[
            pltpu.VMEM((2 * NCH_H, CH), jnp.int32),
            pltpu.VMEM((CH, H), jnp.float32),
            pltpu.VMEM_SHARED((N_PAD, H), jnp.float32),
        ],
        compiler_params=pltpu.CompilerParams(use_tc_tiling_on_sc=False),
    )


def _gru_body(x_ref, agg_ref, wih_ref, whh_ref, bih_ref, bhh_ref,
              lnw_ref, lnb_ref, out_ref):
    x = x_ref[...]
    agg = agg_ref[0] + agg_ref[1]
    gi = jnp.dot(agg, wih_ref[...], preferred_element_type=jnp.float32)
    gi = gi + bih_ref[...]
    gh = jnp.dot(x, whh_ref[...], preferred_element_type=jnp.float32)
    gh = gh + bhh_ref[...]
    r = jax.nn.sigmoid(gi[:, :D] + gh[:, :D])
    z = jax.nn.sigmoid(gi[:, D:2 * D] + gh[:, D:2 * D])
    n = jnp.tanh(gi[:, 2 * D:] + r * gh[:, 2 * D:])
    xn = (1.0 - z) * n + z * x
    mu = jnp.mean(xn, axis=-1, keepdims=True)
    d = xn - mu
    var = jnp.mean(d * d, axis=-1, keepdims=True)
    out_ref[...] = d * lax.rsqrt(var + 1e-5) * lnw_ref[...] + lnb_ref[...]


def _gru_ln(x, agg2, WihT, WhhT, b_ih, b_hh, ln_w, ln_b):
    grid = (N_NODES // BN,)
    return pl.pallas_call(
        _gru_body,
        grid=grid,
        in_specs=[
            pl.BlockSpec((BN, D), lambda i: (i, 0)),
            pl.BlockSpec((2, BN, H), lambda i: (0, i, 0)),
            pl.BlockSpec((H, 3 * D), lambda i: (0, 0)),
            pl.BlockSpec((D, 3 * D), lambda i: (0, 0)),
            pl.BlockSpec((1, 3 * D), lambda i: (0, 0)),
            pl.BlockSpec((1, 3 * D), lambda i: (0, 0)),
            pl.BlockSpec((1, D), lambda i: (0, 0)),
            pl.BlockSpec((1, D), lambda i: (0, 0)),
        ],
        out_specs=pl.BlockSpec((BN, D), lambda i: (i, 0)),
        out_shape=jax.ShapeDtypeStruct((N_NODES, D), jnp.float32),
    )(x, agg2, WihT, WhhT, b_ih, b_hh, ln_w, ln_b)


def kernel(x, edge_index, edge_attr, W1, b1, W2, b2, W_ih, W_hh, b_ih, b_hh,
           ln_w, ln_b):
    row = edge_index[0]
    col = edge_index[1]

    W2b = W2.astype(jnp.bfloat16)
    lane = jnp.arange(H * D, dtype=jnp.int32) // H
    Rsel = (jnp.arange(D, dtype=jnp.int32)[:, None] == lane[None, :])
    Rsel = Rsel.astype(jnp.bfloat16)
    WihT = W_ih.T
    WhhT = W_hh.T

    ea_pad = jnp.pad(edge_attr, ((0, E_PAD - N_EDGES), (0, 0)))
    col_r = jnp.pad(col, (0, E_PAD - N_EDGES)).reshape(2, NW, NCH_H, CH)
    row_r = jnp.pad(row, (0, E_PAD - N_EDGES)).reshape(2, NW, NCH_H, CH)
    row_all = jnp.concatenate([row_r[0], row_r[1]], axis=1)
    zeros = jnp.zeros((N_PAD, H), jnp.float32)

    hj0 = _gather()(x, col_r[0])
    hj1 = _gather()(x, col_r[1])
    b1r = b1.reshape(1, D)
    b2r = b2.reshape(D, H)
    m0 = _edge_messages(ea_pad[:E_HALF], hj0, W1, b1r, W2b, b2r, Rsel, 0)
    m1 = _edge_messages(ea_pad[E_HALF:], hj1, W1, b1r, W2b, b2r, Rsel, E_HALF)
    agg2 = _scatter()(m0, m1, row_all, zeros)
    return _gru_ln(x, agg2, WihT, WhhT, b_ih.reshape(1, 3 * D),
                   b_hh.reshape(1, 3 * D), ln_w.reshape(1, D),
                   ln_b.reshape(1, D))


# speedup vs baseline: 1.0061x; 1.0061x over previous
import functools

import jax
import jax.numpy as jnp
from jax import lax
from jax.experimental import pallas as pl
from jax.experimental.pallas import tpu as pltpu
from jax.experimental.pallas import tpu_sc as plsc

N_NODES = 10000
N_EDGES = 40000
D = 128
H = 32
ED = 16

NW = 32
CH = 128
NCH_H = 5
E_PER_W_H = NCH_H * CH
E_HALF = NW * E_PER_W_H
E_PAD = 2 * E_HALF
N_PAD = 10240
ZROWS = N_PAD // 16
OROWS = N_NODES // 16

BE = 256
BN = 1000


def _gather_body(x_hbm, col_hbm, out_hbm, idx_v, rows_v, sem0, sem1):
    c = lax.axis_index("c")
    s = lax.axis_index("s")
    wid = s * 2 + c
    base = wid * E_PER_W_H
    sems = (sem0, sem1)

    pltpu.sync_copy(col_hbm.at[wid], idx_v)
    cps = [pltpu.async_copy(x_hbm.at[idx_v.at[0]], rows_v.at[0], sem0), None]
    for j in range(NCH_H):
        b = j % 2
        if j + 1 < NCH_H:
            nb = (j + 1) % 2
            cps[nb] = pltpu.async_copy(
                x_hbm.at[idx_v.at[j + 1]], rows_v.at[nb], sems[nb])
        cps[b].wait()
        pltpu.sync_copy(rows_v.at[b], out_hbm.at[pl.ds(base + j * CH, CH)])


@functools.cache
def _gather():
    mesh = plsc.VectorSubcoreMesh(core_axis_name="c", subcore_axis_name="s")
    return pl.kernel(
        _gather_body,
        mesh=mesh,
        out_type=jax.ShapeDtypeStruct((E_HALF, D), jnp.float32),
        scratch_types=[
            pltpu.VMEM((NCH_H, CH), jnp.int32),
            pltpu.VMEM((2, CH, D), jnp.float32),
            pltpu.SemaphoreType.DMA,
            pltpu.SemaphoreType.DMA,
        ],
    )


def _edge_body(ea_ref, hj_ref, w1_ref, b1_ref, w2v_ref, rsel_ref, b2m_ref,
               out_ref, *, e_base):
    ea = ea_ref[...]
    hj = hj_ref[...]
    h = jnp.dot(ea, w1_ref[...], preferred_element_type=jnp.float32)
    h = h + b1_ref[...]
    h = h * jax.nn.sigmoid(h)
    c = jnp.dot(hj.astype(jnp.bfloat16), w2v_ref[...],
                preferred_element_type=jnp.float32)
    h_rep = jnp.dot(h.astype(jnp.bfloat16), rsel_ref[...],
                    preferred_element_type=jnp.float32)
    p = c * h_rep
    w = D * H
    while w > H:
        w //= 2
        p = p[:, :w] + p[:, w:]
    m = p + jnp.dot(hj, b2m_ref[...], preferred_element_type=jnp.float32)
    e0 = e_base + pl.program_id(0) * BE
    eids = e0 + lax.broadcasted_iota(jnp.int32, (BE, H), 0)
    out_ref[...] = jnp.where(eids < N_EDGES, m, 0.0)


def _edge_messages(ea_pad, hj, W1, b1, W2v, Rsel, B2, e_base):
    grid = (E_HALF // BE,)
    blk_ofs = e_base // BE
    return pl.pallas_call(
        functools.partial(_edge_body, e_base=e_base),
        grid=grid,
        in_specs=[
            pl.BlockSpec((BE, ED), lambda i: (i + blk_ofs, 0)),
            pl.BlockSpec((BE, D), lambda i: (i, 0)),
            pl.BlockSpec((ED, D), lambda i: (0, 0)),
            pl.BlockSpec((1, D), lambda i: (0, 0)),
            pl.BlockSpec((D, H * D), lambda i: (0, 0)),
            pl.BlockSpec((D, H * D), lambda i: (0, 0)),
            pl.BlockSpec((D, H), lambda i: (0, 0)),
        ],
        out_specs=pl.BlockSpec((BE, H), lambda i: (i, 0)),
        out_shape=jax.ShapeDtypeStruct((E_HALF, H), jnp.float32),
    )(ea_pad, hj, W1, b1, W2v, Rsel, B2)


def _scatter_body(msg0_hbm, msg1_hbm, row_hbm, zeros_hbm, out_hbm,
                  idx_v, msg_v, shared):
    c = lax.axis_index("c")
    s = lax.axis_index("s")
    wid = s * 2 + c
    pltpu.sync_copy(zeros_hbm.at[pl.ds(s * ZROWS, ZROWS)],
                    shared.at[pl.ds(s * ZROWS, ZROWS)])
    pltpu.sync_copy(row_hbm.at[wid], idx_v)
    plsc.subcore_barrier()

    for j in range(2 * NCH_H):
        src = msg0_hbm if j < NCH_H else msg1_hbm
        off = wid * E_PER_W_H + (j % NCH_H) * CH
        pltpu.sync_copy(src.at[pl.ds(off, CH)], msg_v)
        pltpu.sync_copy(msg_v, shared.at[idx_v.at[j]], add=True)

    plsc.subcore_barrier()
    pltpu.sync_copy(shared.at[pl.ds(s * OROWS, OROWS)],
                    out_hbm.at[c, pl.ds(s * OROWS, OROWS)])


@functools.cache
def _scatter():
    mesh = plsc.VectorSubcoreMesh(core_axis_name="c", subcore_axis_name="s")
    return pl.kernel(
        _scatter_body,
        mesh=mesh,
        out_type=jax.ShapeDtypeStruct((2, N_NODES, H), jnp.float32),
        scratch_types=[
            pltpu.VMEM((2 * NCH_H, CH), jnp.int32),
            pltpu.VMEM((CH, H), jnp.float32),
            pltpu.VMEM_SHARED((N_PAD, H), jnp.float32),
        ],
        compiler_params=pltpu.CompilerParams(use_tc_tiling_on_sc=False),
    )


def _gru_body(x_ref, agg_ref, wih_ref, whh_ref, bih_ref, bhh_ref,
              lnw_ref, lnb_ref, out_ref):
    x = x_ref[...]
    agg = agg_ref[0] + agg_ref[1]
    gi = jnp.dot(agg, wih_ref[...], preferred_element_type=jnp.float32)
    gi = gi + bih_ref[...]
    gh = jnp.dot(x, whh_ref[...], preferred_element_type=jnp.float32)
    gh = gh + bhh_ref[...]
    r = jax.nn.sigmoid(gi[:, :D] + gh[:, :D])
    z = jax.nn.sigmoid(gi[:, D:2 * D] + gh[:, D:2 * D])
    n = jnp.tanh(gi[:, 2 * D:] + r * gh[:, 2 * D:])
    xn = (1.0 - z) * n + z * x
    mu = jnp.mean(xn, axis=-1, keepdims=True)
    d = xn - mu
    var = jnp.mean(d * d, axis=-1, keepdims=True)
    out_ref[...] = d * lax.rsqrt(var + 1e-5) * lnw_ref[...] + lnb_ref[...]


def _gru_ln(x, agg2, WihT, WhhT, b_ih, b_hh, ln_w, ln_b):
    grid = (N_NODES // BN,)
    return pl.pallas_call(
        _gru_body,
        grid=grid,
        in_specs=[
            pl.BlockSpec((BN, D), lambda i: (i, 0)),
            pl.BlockSpec((2, BN, H), lambda i: (0, i, 0)),
            pl.BlockSpec((H, 3 * D), lambda i: (0, 0)),
            pl.BlockSpec((D, 3 * D), lambda i: (0, 0)),
            pl.BlockSpec((1, 3 * D), lambda i: (0, 0)),
            pl.BlockSpec((1, 3 * D), lambda i: (0, 0)),
            pl.BlockSpec((1, D), lambda i: (0, 0)),
            pl.BlockSpec((1, D), lambda i: (0, 0)),
        ],
        out_specs=pl.BlockSpec((BN, D), lambda i: (i, 0)),
        out_shape=jax.ShapeDtypeStruct((N_NODES, D), jnp.float32),
    )(x, agg2, WihT, WhhT, b_ih, b_hh, ln_w, ln_b)


def kernel(x, edge_index, edge_attr, W1, b1, W2, b2, W_ih, W_hh, b_ih, b_hh,
           ln_w, ln_b):
    row = edge_index[0]
    col = edge_index[1]

    W2v = W2.reshape(D, D, H).transpose(1, 0, 2).reshape(D, H * D)
    W2v = W2v.astype(jnp.bfloat16)
    lane = jnp.arange(H * D, dtype=jnp.int32) // H
    Rsel = (jnp.arange(D, dtype=jnp.int32)[:, None] == lane[None, :])
    Rsel = Rsel.astype(jnp.bfloat16)
    WihT = W_ih.T
    WhhT = W_hh.T

    ea_pad = jnp.pad(edge_attr, ((0, E_PAD - N_EDGES), (0, 0)))
    col_r = jnp.pad(col, (0, E_PAD - N_EDGES)).reshape(2, NW, NCH_H, CH)
    row_r = jnp.pad(row, (0, E_PAD - N_EDGES)).reshape(2, NW, NCH_H, CH)
    row_all = jnp.concatenate([row_r[0], row_r[1]], axis=1)
    zeros = jnp.zeros((N_PAD, H), jnp.float32)

    hj0 = _gather()(x, col_r[0])
    hj1 = _gather()(x, col_r[1])
    b1r = b1.reshape(1, D)
    B2 = b2.reshape(D, H)
    m0 = _edge_messages(ea_pad, hj0, W1, b1r, W2v, Rsel, B2, 0)
    m1 = _edge_messages(ea_pad, hj1, W1, b1r, W2v, Rsel, B2, E_HALF)
    agg2 = _scatter()(m0, m1, row_all, zeros)
    return _gru_ln(x, agg2, WihT, WhhT, b_ih.reshape(1, 3 * D),
                   b_hh.reshape(1, 3 * D), ln_w.reshape(1, D),
                   ln_b.reshape(1, D))


# speedup vs baseline: 1.1643x; 1.1573x over previous
import functools

import jax
import jax.numpy as jnp
from jax import lax
from jax.experimental import pallas as pl
from jax.experimental.pallas import tpu as pltpu
from jax.experimental.pallas import tpu_sc as plsc

N_NODES = 10000
N_EDGES = 40000
D = 128
H = 32
ED = 16

NW = 32
CH = 128
NCH_H = 5
E_PER_W_H = NCH_H * CH
E_HALF = NW * E_PER_W_H
H1_BASE = N_EDGES - E_HALF
N_PAD = 10240
ZROWS = N_PAD // 16
OROWS = N_NODES // 16

BE = 320
BN = 1000


def _gather_body(x_hbm, col_hbm, out_hbm, idx_v, rows_v, sem0, sem1):
    c = lax.axis_index("c")
    s = lax.axis_index("s")
    wid = s * 2 + c
    base = wid * E_PER_W_H
    sems = (sem0, sem1)

    pltpu.sync_copy(col_hbm.at[wid], idx_v)
    cps = [pltpu.async_copy(x_hbm.at[idx_v.at[0]], rows_v.at[0], sem0), None]
    for j in range(NCH_H):
        b = j % 2
        if j + 1 < NCH_H:
            nb = (j + 1) % 2
            cps[nb] = pltpu.async_copy(
                x_hbm.at[idx_v.at[j + 1]], rows_v.at[nb], sems[nb])
        cps[b].wait()
        pltpu.sync_copy(rows_v.at[b], out_hbm.at[pl.ds(base + j * CH, CH)])


@functools.cache
def _gather():
    mesh = plsc.VectorSubcoreMesh(core_axis_name="c", subcore_axis_name="s")
    return pl.kernel(
        _gather_body,
        mesh=mesh,
        out_type=jax.ShapeDtypeStruct((E_HALF, D), jnp.float32),
        scratch_types=[
            pltpu.VMEM((NCH_H, CH), jnp.int32),
            pltpu.VMEM((2, CH, D), jnp.float32),
            pltpu.SemaphoreType.DMA,
            pltpu.SemaphoreType.DMA,
        ],
    )


def _edge_body(ea_ref, hj_ref, w1_ref, b1_ref, w2v_ref, rsel_ref, b2m_ref,
               out_ref, *, e_base, keep_lo):
    ea = ea_ref[...]
    hj = hj_ref[...]
    h = jnp.dot(ea, w1_ref[...], preferred_element_type=jnp.float32)
    h = h + b1_ref[...]
    h = h * jax.nn.sigmoid(h)
    c = jnp.dot(hj.astype(jnp.bfloat16), w2v_ref[...],
                preferred_element_type=jnp.float32)
    h_rep = jnp.dot(h.astype(jnp.bfloat16), rsel_ref[...],
                    preferred_element_type=jnp.float32)
    p = c * h_rep
    w = D * H
    while w > H:
        w //= 2
        p = p[:, :w] + p[:, w:]
    m = p + jnp.dot(hj, b2m_ref[...], preferred_element_type=jnp.float32)
    e0 = e_base + pl.program_id(0) * BE
    eids = e0 + lax.broadcasted_iota(jnp.int32, (BE, H), 0)
    out_ref[...] = jnp.where(eids >= keep_lo, m, 0.0)


def _edge_messages(ea, hj, W1, b1, W2v, Rsel, B2, e_base, keep_lo):
    grid = (E_HALF // BE,)
    blk_ofs = e_base // BE
    return pl.pallas_call(
        functools.partial(_edge_body, e_base=e_base, keep_lo=keep_lo),
        grid=grid,
        in_specs=[
            pl.BlockSpec((BE, ED), lambda i: (i + blk_ofs, 0)),
            pl.BlockSpec((BE, D), lambda i: (i, 0)),
            pl.BlockSpec((ED, D), lambda i: (0, 0)),
            pl.BlockSpec((1, D), lambda i: (0, 0)),
            pl.BlockSpec((D, H * D), lambda i: (0, 0)),
            pl.BlockSpec((D, H * D), lambda i: (0, 0)),
            pl.BlockSpec((D, H), lambda i: (0, 0)),
        ],
        out_specs=pl.BlockSpec((BE, H), lambda i: (i, 0)),
        out_shape=jax.ShapeDtypeStruct((E_HALF, H), jnp.float32),
    )(ea, hj, W1, b1, W2v, Rsel, B2)


def _scatter_body(msg0_hbm, msg1_hbm, row_hbm, zeros_hbm, out_hbm,
                  idx_v, msg_v, shared):
    c = lax.axis_index("c")
    s = lax.axis_index("s")
    wid = s * 2 + c
    pltpu.sync_copy(zeros_hbm.at[pl.ds(s * ZROWS, ZROWS)],
                    shared.at[pl.ds(s * ZROWS, ZROWS)])
    pltpu.sync_copy(row_hbm.at[wid], idx_v)
    plsc.subcore_barrier()

    for j in range(2 * NCH_H):
        src = msg0_hbm if j < NCH_H else msg1_hbm
        off = wid * E_PER_W_H + (j % NCH_H) * CH
        pltpu.sync_copy(src.at[pl.ds(off, CH)], msg_v)
        pltpu.sync_copy(msg_v, shared.at[idx_v.at[j]], add=True)

    plsc.subcore_barrier()
    pltpu.sync_copy(shared.at[pl.ds(s * OROWS, OROWS)],
                    out_hbm.at[c, pl.ds(s * OROWS, OROWS)])


@functools.cache
def _scatter():
    mesh = plsc.VectorSubcoreMesh(core_axis_name="c", subcore_axis_name="s")
    return pl.kernel(
        _scatter_body,
        mesh=mesh,
        out_type=jax.ShapeDtypeStruct((2, N_NODES, H), jnp.float32),
        scratch_types=[
            pltpu.VMEM((2 * NCH_H, CH), jnp.int32),
            pltpu.VMEM((CH, H), jnp.float32),
            pltpu.VMEM_SHARED((N_PAD, H), jnp.float32),
        ],
        compiler_params=pltpu.CompilerParams(use_tc_tiling_on_sc=False),
    )


def _gru_body(x_ref, agg_ref, wih_ref, whh_ref, bih_ref, bhh_ref,
              lnw_ref, lnb_ref, out_ref):
    x = x_ref[...]
    agg = agg_ref[0] + agg_ref[1]
    gi = jnp.dot(agg, wih_ref[...], preferred_element_type=jnp.float32)
    gi = gi + bih_ref[...]
    gh = jnp.dot(x, whh_ref[...], preferred_element_type=jnp.float32)
    gh = gh + bhh_ref[...]
    r = jax.nn.sigmoid(gi[:, :D] + gh[:, :D])
    z = jax.nn.sigmoid(gi[:, D:2 * D] + gh[:, D:2 * D])
    n = jnp.tanh(gi[:, 2 * D:] + r * gh[:, 2 * D:])
    xn = (1.0 - z) * n + z * x
    mu = jnp.mean(xn, axis=-1, keepdims=True)
    d = xn - mu
    var = jnp.mean(d * d, axis=-1, keepdims=True)
    out_ref[...] = d * lax.rsqrt(var + 1e-5) * lnw_ref[...] + lnb_ref[...]


def _gru_ln(x, agg2, WihT, WhhT, b_ih, b_hh, ln_w, ln_b):
    grid = (N_NODES // BN,)
    return pl.pallas_call(
        _gru_body,
        grid=grid,
        in_specs=[
            pl.BlockSpec((BN, D), lambda i: (i, 0)),
            pl.BlockSpec((2, BN, H), lambda i: (0, i, 0)),
            pl.BlockSpec((H, 3 * D), lambda i: (0, 0)),
            pl.BlockSpec((D, 3 * D), lambda i: (0, 0)),
            pl.BlockSpec((1, 3 * D), lambda i: (0, 0)),
            pl.BlockSpec((1, 3 * D), lambda i: (0, 0)),
            pl.BlockSpec((1, D), lambda i: (0, 0)),
            pl.BlockSpec((1, D), lambda i: (0, 0)),
        ],
        out_specs=pl.BlockSpec((BN, D), lambda i: (i, 0)),
        out_shape=jax.ShapeDtypeStruct((N_NODES, D), jnp.float32),
    )(x, agg2, WihT, WhhT, b_ih, b_hh, ln_w, ln_b)


def kernel(x, edge_index, edge_attr, W1, b1, W2, b2, W_ih, W_hh, b_ih, b_hh,
           ln_w, ln_b):
    row = edge_index[0]
    col = edge_index[1]

    W2v = W2.reshape(D, D, H).transpose(1, 0, 2).reshape(D, H * D)
    W2v = W2v.astype(jnp.bfloat16)
    lane = jnp.arange(H * D, dtype=jnp.int32) // H
    Rsel = (jnp.arange(D, dtype=jnp.int32)[:, None] == lane[None, :])
    Rsel = Rsel.astype(jnp.bfloat16)
    WihT = W_ih.T
    WhhT = W_hh.T

    col0 = col[:E_HALF].reshape(NW, NCH_H, CH)
    col1 = col[H1_BASE:].reshape(NW, NCH_H, CH)
    row_all = jnp.concatenate([row[:E_HALF].reshape(NW, NCH_H, CH),
                               row[H1_BASE:].reshape(NW, NCH_H, CH)],
                              axis=1)
    zeros = jnp.zeros((N_PAD, H), jnp.float32)

    hj0 = _gather()(x, col0)
    hj1 = _gather()(x, col1)
    b1r = b1.reshape(1, D)
    B2 = b2.reshape(D, H)
    m0 = _edge_messages(edge_attr, hj0, W1, b1r, W2v, Rsel, B2, 0, 0)
    m1 = _edge_messages(edge_attr, hj1, W1, b1r, W2v, Rsel, B2, H1_BASE,
                        E_HALF)
    agg2 = _scatter()(m0, m1, row_all, zeros)
    return _gru_ln(x, agg2, WihT, WhhT, b_ih.reshape(1, 3 * D),
                   b_hh.reshape(1, 3 * D), ln_w.reshape(1, D),
                   ln_b.reshape(1, D))
